# baseline (speedup 1.0000x reference)
import functools

import jax
import jax.numpy as jnp
from jax import lax
from jax.experimental import pallas as pl
from jax.experimental.pallas import tpu as pltpu
from jax.experimental.pallas import tpu_sc as plsc

_NC = 2
_NS = 16
_NW = _NC * _NS
_L = 16


@functools.lru_cache(maxsize=None)
def _build(S, B, D, V, P):
    rows_w = S // _NW
    nchunk = rows_w // P
    mesh = plsc.VectorSubcoreMesh(
        core_axis_name="c", subcore_axis_name="s",
        num_cores=_NC, num_subcores=_NS,
    )

    @functools.partial(
        pl.kernel,
        out_type=jax.ShapeDtypeStruct((S, B, D), jnp.float32),
        mesh=mesh,
        scratch_types=[
            pltpu.VMEM((rows_w,), jnp.int32),
            [pltpu.VMEM((P, B, D), jnp.float32) for _ in range(3)],
            [pltpu.VMEM((P, 1, D), jnp.float32) for _ in range(3)],
            [pltpu.SemaphoreType.DMA for _ in range(3)],
            [pltpu.SemaphoreType.DMA for _ in range(3)],
            [pltpu.SemaphoreType.DMA for _ in range(3)],
        ],
    )
    def sc_add(x_hbm, i_hbm, pe_hbm, out_hbm, idx_v, xbuf, pebuf, insem, gsem, osem):
        wid = lax.axis_index("s") * _NC + lax.axis_index("c")
        base = wid * rows_w
        idx_cp = pltpu.make_async_copy(
            i_hbm.at[pl.ds(base, rows_w)], idx_v, gsem[0])

        def x_copy(c, t):
            coff = pl.multiple_of(c * P, 8)
            return pltpu.make_async_copy(
                x_hbm.at[pl.ds(base + coff, P)], xbuf[t], insem[t])

        def pe_copy(c, t):
            coff = pl.multiple_of(c * P, 8)
            return pltpu.make_async_copy(
                pe_hbm.at[idx_v.at[pl.ds(coff, P)]], pebuf[t], gsem[t])

        def in_copies(c, t):
            return (x_copy(c, t), pe_copy(c, t))

        SPLITS = ((0, 2), (2, 6))

        def out_half(c, t, h):
            coff = pl.multiple_of(c * P, 8)
            lo, n = SPLITS[h]
            return pltpu.make_async_copy(
                xbuf[t].at[pl.ds(lo, n)],
                out_hbm.at[pl.ds(base + coff + lo, n)], osem[t])

        def wait_out(c, t):
            for h in range(len(SPLITS)):
                out_half(c, t, h).wait()

        def compute(t, h):
            xb, pb = xbuf[t], pebuf[t]
            lo, n = SPLITS[h]

            @plsc.parallel_loop(0, D // _L, 1, unroll=8)
            def _kbody(k, xb=xb, pb=pb, lo=lo, n=n):
                sl = pl.ds(k * _L, _L)
                for p in range(lo, lo + n):
                    pv = pb[p, 0, sl]
                    for b in range(B):
                        xb[p, b, sl] += pv

        def process(c, t):
            for cp in in_copies(c, t):
                cp.wait()
            for h in range(len(SPLITS)):
                compute(t, h)
                out_half(c, t, h).start()

        idx_cp.start()
        x_copy(0, 0).start()
        x_copy(1, 1).start()
        x_copy(2, 2).start()
        idx_cp.wait()
        pe_copy(0, 0).start()
        pe_copy(1, 1).start()
        pe_copy(2, 2).start()
        process(0, 0)

        def triple(j, carry):
            c = 3 * j + 1
            for u, t in ((0, 1), (1, 2), (2, 0)):
                wait_out(c + u - 1, (t + 2) % 3)
                for cp in in_copies(c + u + 2, (t + 2) % 3):
                    cp.start()
                process(c + u, t)
            return carry

        lax.fori_loop(0, (nchunk - 4) // 3, triple, 0)
        wait_out(nchunk - 4, 0)
        for cp in in_copies(nchunk - 1, 0):
            cp.start()
        process(nchunk - 3, 1)
        process(nchunk - 2, 2)
        process(nchunk - 1, 0)
        wait_out(nchunk - 3, 1)
        wait_out(nchunk - 2, 2)
        wait_out(nchunk - 1, 0)

    return sc_add


def kernel(x, i, pe):
    S, B, D = x.shape
    V = pe.shape[0]
    P = 8
    return _build(S, B, D, V, P)(x, i.astype(jnp.int32), pe)


# speedup vs baseline: 1.0190x; 1.0190x over previous
import functools

import jax
import jax.numpy as jnp
from jax import lax
from jax.experimental import pallas as pl
from jax.experimental.pallas import tpu as pltpu
from jax.experimental.pallas import tpu_sc as plsc

_NC = 2
_NS = 16
_NW = _NC * _NS
_L = 16


@functools.lru_cache(maxsize=None)
def _build(S, B, D, V, P):
    rows_w = S // _NW
    nchunk = rows_w // P
    mesh = plsc.VectorSubcoreMesh(
        core_axis_name="c", subcore_axis_name="s",
        num_cores=_NC, num_subcores=_NS,
    )

    @functools.partial(
        pl.kernel,
        out_type=jax.ShapeDtypeStruct((S, B, D), jnp.float32),
        mesh=mesh,
        scratch_types=[
            pltpu.VMEM((rows_w,), jnp.int32),
            [pltpu.VMEM((P, B, D), jnp.float32) for _ in range(3)],
            [pltpu.VMEM((P, 1, D), jnp.float32) for _ in range(3)],
            [pltpu.SemaphoreType.DMA for _ in range(3)],
            [pltpu.SemaphoreType.DMA for _ in range(3)],
            [pltpu.SemaphoreType.DMA for _ in range(3)],
        ],
    )
    def sc_add(x_hbm, i_hbm, pe_hbm, out_hbm, idx_v, xbuf, pebuf, insem, gsem, osem):
        wid = lax.axis_index("s") * _NC + lax.axis_index("c")
        base = wid * rows_w
        idx_cp = pltpu.make_async_copy(
            i_hbm.at[pl.ds(base, rows_w)], idx_v, gsem[0])

        def x_copy(c, t):
            coff = pl.multiple_of(c * P, 8)
            return pltpu.make_async_copy(
                x_hbm.at[pl.ds(base + coff, P)], xbuf[t], insem[t])

        def pe_copy(c, t):
            coff = pl.multiple_of(c * P, 8)
            return pltpu.make_async_copy(
                pe_hbm.at[idx_v.at[pl.ds(coff, P)]], pebuf[t], gsem[t])

        def in_copies(c, t):
            return (x_copy(c, t), pe_copy(c, t))

        SPLITS = ((0, 4), (4, 4))

        def out_half(c, t, h):
            coff = pl.multiple_of(c * P, 8)
            lo, n = SPLITS[h]
            return pltpu.make_async_copy(
                xbuf[t].at[pl.ds(lo, n)],
                out_hbm.at[pl.ds(base + coff + lo, n)], osem[t])

        def wait_out(c, t):
            for h in range(len(SPLITS)):
                out_half(c, t, h).wait()

        def compute(t, h):
            xb, pb = xbuf[t], pebuf[t]
            lo, n = SPLITS[h]

            @plsc.parallel_loop(0, D // _L, 1, unroll=8)
            def _kbody(k, xb=xb, pb=pb, lo=lo, n=n):
                sl = pl.ds(k * _L, _L)
                for p in range(lo, lo + n):
                    pv = pb[p, 0, sl]
                    for b in range(B):
                        xb[p, b, sl] += pv

        def process(c, t):
            for cp in in_copies(c, t):
                cp.wait()
            for h in range(len(SPLITS)):
                compute(t, h)
                out_half(c, t, h).start()

        idx_cp.start()
        x_copy(0, 0).start()
        x_copy(1, 1).start()
        x_copy(2, 2).start()
        idx_cp.wait()
        pe_copy(0, 0).start()
        pe_copy(1, 1).start()
        pe_copy(2, 2).start()
        process(0, 0)

        def triple(j, carry):
            c = 3 * j + 1
            for u, t in ((0, 1), (1, 2), (2, 0)):
                wait_out(c + u - 1, (t + 2) % 3)
                for cp in in_copies(c + u + 2, (t + 2) % 3):
                    cp.start()
                process(c + u, t)
            return carry

        lax.fori_loop(0, (nchunk - 4) // 3, triple, 0)
        wait_out(nchunk - 4, 0)
        for cp in in_copies(nchunk - 1, 0):
            cp.start()
        process(nchunk - 3, 1)
        process(nchunk - 2, 2)
        process(nchunk - 1, 0)
        wait_out(nchunk - 3, 1)
        wait_out(nchunk - 2, 2)
        wait_out(nchunk - 1, 0)

    return sc_add


def kernel(x, i, pe):
    S, B, D = x.shape
    V = pe.shape[0]
    P = 8
    return _build(S, B, D, V, P)(x, i.astype(jnp.int32), pe)
